# baseline (speedup 1.0000x reference)
import functools

import jax
import jax.numpy as jnp
from jax import lax
from jax.experimental import pallas as pl
from jax.experimental.pallas import tpu as pltpu
from jax.experimental.pallas import tpu_sc as plsc

D = 64
NB = 16384
NS = 50
NW = 32
B_PER_W = NB // NW
NBBLK = NB // 128
BBLK_PER_W = NBBLK // NW
N_UNITS = BBLK_PER_W * NS
SLAB = B_PER_W * NS


def _sc_embedding_gather(idx_flat, weight):
    mesh = plsc.VectorSubcoreMesh(core_axis_name="c", subcore_axis_name="s")

    @functools.partial(
        pl.kernel,
        mesh=mesh,
        out_type=jax.ShapeDtypeStruct((NS, D // 8, NBBLK, 8, 128),
                                      jnp.float32),
        scratch_types=[
            pltpu.VMEM((128,), jnp.int32),
            pltpu.VMEM((128,), jnp.int32),
            pltpu.VMEM((128, D), jnp.float32),
            pltpu.VMEM((128, D), jnp.float32),
            pltpu.VMEM((D // 8, 8, 128), jnp.float32),
            pltpu.VMEM((D // 8, 8, 128), jnp.float32),
            pltpu.SemaphoreType.DMA,
            pltpu.SemaphoreType.DMA,
            pltpu.SemaphoreType.DMA,
            pltpu.SemaphoreType.DMA,
            pltpu.SemaphoreType.DMA,
            pltpu.SemaphoreType.DMA,
        ],
        compiler_params=pltpu.CompilerParams(
            use_tc_tiling_on_sc=False, needs_layout_passes=False),
    )
    def k(table_hbm, idx_hbm, out_hbm, gidx_a, gidx_b, rows_a,
          rows_b, rt_a, rt_b, sem_ia, sem_ib, sem_ga, sem_gb, sem_sa,
          sem_sb):
        wid = lax.axis_index("s") * 2 + lax.axis_index("c")

        iota = lax.iota(jnp.int32, 16)

        def unit_sb(u):
            return u % NS, u // NS

        def load_gidx(u, gidx, sem):
            s, bb = unit_sb(u)
            pltpu.async_copy(
                idx_hbm.at[pl.ds(s * NB + wid * B_PER_W + bb * 128, 128)],
                gidx, sem)

        def wait_gidx(u, gidx, sem):
            s, bb = unit_sb(u)
            pltpu.make_async_copy(
                idx_hbm.at[pl.ds(s * NB + wid * B_PER_W + bb * 128, 128)],
                gidx, sem).wait()

        def gather(gidx, rows, sem):
            pltpu.async_copy(table_hbm.at[gidx], rows, sem)

        def wait_gather(gidx, rows, sem):
            pltpu.make_async_copy(table_hbm.at[gidx], rows, sem).wait()

        bases = [iota + bg * 16 for bg in range(8)]

        def transpose(rows, rt):
            @plsc.parallel_loop(0, D, unroll=4)
            def _(d):
                col = jnp.full((16,), d, jnp.int32)
                dhi = d // 8
                dlo = d % 8
                for bg in range(8):
                    v = plsc.load_gather(rows, [bases[bg], col])
                    rt[dhi, dlo, pl.ds(bg * 16, 16)] = v

        def store(u, rt, sem):
            s, bb = unit_sb(u)
            pltpu.async_copy(rt, out_hbm.at[s, :, wid * BBLK_PER_W + bb],
                             sem)

        def wait_store(u, rt, sem):
            s, bb = unit_sb(u)
            pltpu.make_async_copy(
                rt, out_hbm.at[s, :, wid * BBLK_PER_W + bb], sem).wait()

        load_gidx(0, gidx_a, sem_ia)
        load_gidx(1, gidx_b, sem_ib)
        wait_gidx(0, gidx_a, sem_ia)
        gather(gidx_a, rows_a, sem_ga)
        wait_gidx(1, gidx_b, sem_ib)
        gather(gidx_b, rows_b, sem_gb)

        def body(i, carry):
            u0 = 2 * i
            u1 = u0 + 1
            wait_gather(gidx_a, rows_a, sem_ga)

            @pl.when(u0 + 2 < N_UNITS)
            def _():
                load_gidx(u0 + 2, gidx_a, sem_ia)

            @pl.when(i > 0)
            def _():
                wait_store(u0 - 2, rt_a, sem_sa)

            transpose(rows_a, rt_a)
            store(u0, rt_a, sem_sa)

            @pl.when(u0 + 2 < N_UNITS)
            def _():
                wait_gidx(u0 + 2, gidx_a, sem_ia)
                gather(gidx_a, rows_a, sem_ga)

            wait_gather(gidx_b, rows_b, sem_gb)

            @pl.when(u1 + 2 < N_UNITS)
            def _():
                load_gidx(u1 + 2, gidx_b, sem_ib)

            @pl.when(i > 0)
            def _():
                wait_store(u1 - 2, rt_b, sem_sb)

            transpose(rows_b, rt_b)
            store(u1, rt_b, sem_sb)

            @pl.when(u1 + 2 < N_UNITS)
            def _():
                wait_gidx(u1 + 2, gidx_b, sem_ib)
                gather(gidx_b, rows_b, sem_gb)

            return carry

        lax.fori_loop(0, N_UNITS // 2, body, 0)
        wait_store(N_UNITS - 2, rt_a, sem_sa)
        wait_store(N_UNITS - 1, rt_b, sem_sb)

    return k(weight, idx_flat)


NV = 1000000
TBLK = 2048


def _tc_relayout_table(wt):
    grid = (NV + TBLK - 1) // TBLK

    def body(wt_ref, out_ref):
        x = wt_ref[...]
        eye = jnp.eye(D, dtype=jnp.float32)
        y = lax.dot_general(x, eye, (((0,), (0,)), ((), ())),
                            precision=lax.Precision.HIGHEST,
                            preferred_element_type=jnp.float32)
        out_ref[:, 0:D] = y
        out_ref[:, D:128] = jnp.zeros((TBLK, 128 - D), jnp.float32)

    return pl.pallas_call(
        body,
        grid=(grid,),
        in_specs=[pl.BlockSpec((D, TBLK), lambda i: (0, i))],
        out_specs=pl.BlockSpec((TBLK, 128), lambda i: (i, 0)),
        out_shape=jax.ShapeDtypeStruct((NV, 128), jnp.float32),
    )(wt)


def kernel(input, weight):
    idx_flat = input.T.reshape(-1).astype(jnp.int32) * 2
    w_pad = _tc_relayout_table(weight.T)
    w2m = jnp.reshape(w_pad, (2 * NV, D))
    out5 = _sc_embedding_gather(idx_flat, w2m)
    return jnp.transpose(out5, (2, 4, 0, 1, 3)).reshape(NB, NS, D)


# speedup vs baseline: 1.1318x; 1.1318x over previous
import functools

import jax
import jax.numpy as jnp
from jax import lax
from jax.experimental import pallas as pl
from jax.experimental.pallas import tpu as pltpu
from jax.experimental.pallas import tpu_sc as plsc

D = 64
NB = 16384
NS = 50
NW = 32
B_PER_W = NB // NW
NBBLK = NB // 128
BBLK_PER_W = NBBLK // NW
N_UNITS = BBLK_PER_W * NS
SLAB = B_PER_W * NS


def _sc_embedding_gather(idx_flat, weight):
    mesh = plsc.VectorSubcoreMesh(core_axis_name="c", subcore_axis_name="s")

    @functools.partial(
        pl.kernel,
        mesh=mesh,
        out_type=jax.ShapeDtypeStruct((NS, D // 8, NBBLK, 8, 128),
                                      jnp.float32),
        scratch_types=[
            pltpu.VMEM((128,), jnp.int32),
            pltpu.VMEM((128,), jnp.int32),
            pltpu.VMEM((128, D), jnp.float32),
            pltpu.VMEM((128, D), jnp.float32),
            pltpu.VMEM((D // 8, 8, 128), jnp.float32),
            pltpu.VMEM((D // 8, 8, 128), jnp.float32),
            pltpu.SemaphoreType.DMA,
            pltpu.SemaphoreType.DMA,
            pltpu.SemaphoreType.DMA,
            pltpu.SemaphoreType.DMA,
            pltpu.SemaphoreType.DMA,
            pltpu.SemaphoreType.DMA,
        ],
        compiler_params=pltpu.CompilerParams(
            use_tc_tiling_on_sc=False, needs_layout_passes=False),
    )
    def k(table_hbm, idx_hbm, out_hbm, gidx_a, gidx_b, rows_a,
          rows_b, rt_a, rt_b, sem_ia, sem_ib, sem_ga, sem_gb, sem_sa,
          sem_sb):
        wid = lax.axis_index("s") * 2 + lax.axis_index("c")

        iota = lax.iota(jnp.int32, 16)

        def unit_sb(u):
            return u % NS, u // NS

        def load_gidx(u, gidx, sem):
            s, bb = unit_sb(u)
            pltpu.async_copy(
                idx_hbm.at[pl.ds(s * NB + wid * B_PER_W + bb * 128, 128)],
                gidx, sem)

        def wait_gidx(u, gidx, sem):
            s, bb = unit_sb(u)
            pltpu.make_async_copy(
                idx_hbm.at[pl.ds(s * NB + wid * B_PER_W + bb * 128, 128)],
                gidx, sem).wait()

        def gather(gidx, rows, sem):
            pltpu.async_copy(table_hbm.at[gidx], rows, sem)

        def wait_gather(gidx, rows, sem):
            pltpu.make_async_copy(table_hbm.at[gidx], rows, sem).wait()

        bases = [iota + bg * 16 for bg in range(8)]

        def transpose(rows, rt):
            @plsc.parallel_loop(0, D, unroll=4)
            def _(d):
                col = jnp.full((16,), d, jnp.int32)
                dhi = d // 8
                dlo = d % 8
                for bg in range(8):
                    v = plsc.load_gather(rows, [bases[bg], col])
                    rt[dhi, dlo, pl.ds(bg * 16, 16)] = v

        def store(u, rt, sem):
            s, bb = unit_sb(u)
            pltpu.async_copy(rt, out_hbm.at[s, :, wid * BBLK_PER_W + bb],
                             sem)

        def wait_store(u, rt, sem):
            s, bb = unit_sb(u)
            pltpu.make_async_copy(
                rt, out_hbm.at[s, :, wid * BBLK_PER_W + bb], sem).wait()

        load_gidx(0, gidx_a, sem_ia)
        load_gidx(1, gidx_b, sem_ib)
        wait_gidx(0, gidx_a, sem_ia)
        gather(gidx_a, rows_a, sem_ga)
        wait_gidx(1, gidx_b, sem_ib)
        gather(gidx_b, rows_b, sem_gb)

        def body(i, carry):
            u0 = 2 * i
            u1 = u0 + 1
            wait_gather(gidx_a, rows_a, sem_ga)

            @pl.when(u0 + 2 < N_UNITS)
            def _():
                load_gidx(u0 + 2, gidx_a, sem_ia)

            @pl.when(i > 0)
            def _():
                wait_store(u0 - 2, rt_a, sem_sa)

            transpose(rows_a, rt_a)
            store(u0, rt_a, sem_sa)

            @pl.when(u0 + 2 < N_UNITS)
            def _():
                wait_gidx(u0 + 2, gidx_a, sem_ia)
                gather(gidx_a, rows_a, sem_ga)

            wait_gather(gidx_b, rows_b, sem_gb)

            @pl.when(u1 + 2 < N_UNITS)
            def _():
                load_gidx(u1 + 2, gidx_b, sem_ib)

            @pl.when(i > 0)
            def _():
                wait_store(u1 - 2, rt_b, sem_sb)

            transpose(rows_b, rt_b)
            store(u1, rt_b, sem_sb)

            @pl.when(u1 + 2 < N_UNITS)
            def _():
                wait_gidx(u1 + 2, gidx_b, sem_ib)
                gather(gidx_b, rows_b, sem_gb)

            return carry

        lax.fori_loop(0, N_UNITS // 2, body, 0)
        wait_store(N_UNITS - 2, rt_a, sem_sa)
        wait_store(N_UNITS - 1, rt_b, sem_sb)

    return k(weight, idx_flat)


NV = 1000000
TBLK = 2048


def _tc_relayout_table(wt):
    grid = (NV + TBLK - 1) // TBLK

    def body(wt_ref, out_ref):
        x = wt_ref[...]
        y = jnp.transpose(x)
        out_ref[:, 0:D] = y
        out_ref[:, D:128] = jnp.zeros((TBLK, 128 - D), jnp.float32)

    return pl.pallas_call(
        body,
        grid=(grid,),
        in_specs=[pl.BlockSpec((D, TBLK), lambda i: (0, i))],
        out_specs=pl.BlockSpec((TBLK, 128), lambda i: (i, 0)),
        out_shape=jax.ShapeDtypeStruct((NV, 128), jnp.float32),
    )(wt)


def kernel(input, weight):
    idx_flat = input.T.reshape(-1).astype(jnp.int32) * 2
    w_pad = _tc_relayout_table(weight.T)
    w2m = jnp.reshape(w_pad, (2 * NV, D))
    out5 = _sc_embedding_gather(idx_flat, w2m)
    return jnp.transpose(out5, (2, 4, 0, 1, 3)).reshape(NB, NS, D)


# speedup vs baseline: 1.1345x; 1.0024x over previous
import functools

import jax
import jax.numpy as jnp
from jax import lax
from jax.experimental import pallas as pl
from jax.experimental.pallas import tpu as pltpu
from jax.experimental.pallas import tpu_sc as plsc

D = 64
NB = 16384
NS = 50
NW = 32
B_PER_W = NB // NW
NBBLK = NB // 128
BBLK_PER_W = NBBLK // NW
UBLK = 256
UB_PER_W = B_PER_W // UBLK
N_UNITS = UB_PER_W * NS


def _sc_embedding_gather(idx_flat, weight):
    mesh = plsc.VectorSubcoreMesh(core_axis_name="c", subcore_axis_name="s")

    @functools.partial(
        pl.kernel,
        mesh=mesh,
        out_type=jax.ShapeDtypeStruct((NS, D // 8, NBBLK, 8, 128),
                                      jnp.float32),
        scratch_types=[
            pltpu.VMEM((UBLK,), jnp.int32),
            pltpu.VMEM((UBLK,), jnp.int32),
            pltpu.VMEM((UBLK, D), jnp.float32),
            pltpu.VMEM((UBLK, D), jnp.float32),
            pltpu.VMEM((D // 8, UBLK // 128, 8, 128), jnp.float32),
            pltpu.VMEM((D // 8, UBLK // 128, 8, 128), jnp.float32),
            pltpu.SemaphoreType.DMA,
            pltpu.SemaphoreType.DMA,
            pltpu.SemaphoreType.DMA,
            pltpu.SemaphoreType.DMA,
            pltpu.SemaphoreType.DMA,
            pltpu.SemaphoreType.DMA,
        ],
        compiler_params=pltpu.CompilerParams(
            use_tc_tiling_on_sc=False, needs_layout_passes=False),
    )
    def k(table_hbm, idx_hbm, out_hbm, gidx_a, gidx_b, rows_a,
          rows_b, rt_a, rt_b, sem_ia, sem_ib, sem_ga, sem_gb, sem_sa,
          sem_sb):
        wid = lax.axis_index("s") * 2 + lax.axis_index("c")

        iota = lax.iota(jnp.int32, 16)

        def unit_sb(u):
            return u % NS, u // NS

        def load_gidx(u, gidx, sem):
            s, bb = unit_sb(u)
            pltpu.async_copy(
                idx_hbm.at[pl.ds(s * NB + wid * B_PER_W + bb * UBLK, UBLK)],
                gidx, sem)

        def wait_gidx(u, gidx, sem):
            s, bb = unit_sb(u)
            pltpu.make_async_copy(
                idx_hbm.at[pl.ds(s * NB + wid * B_PER_W + bb * UBLK, UBLK)],
                gidx, sem).wait()

        def gather(gidx, rows, sem):
            pltpu.async_copy(table_hbm.at[gidx], rows, sem)

        def wait_gather(gidx, rows, sem):
            pltpu.make_async_copy(table_hbm.at[gidx], rows, sem).wait()

        bases = [iota + bg * 16 for bg in range(UBLK // 16)]

        def transpose(rows, rt):
            @plsc.parallel_loop(0, D, unroll=4)
            def _(d):
                col = jnp.full((16,), d, jnp.int32)
                dhi = d // 8
                dlo = d % 8
                for bg in range(UBLK // 16):
                    v = plsc.load_gather(rows, [bases[bg], col])
                    rt[dhi, bg // 8, dlo, pl.ds((bg % 8) * 16, 16)] = v

        def out_slice(u):
            s, bb = unit_sb(u)
            nblk = UBLK // 128
            return out_hbm.at[s, :, pl.ds(wid * BBLK_PER_W + bb * nblk,
                                          nblk)]

        def store(u, rt, sem):
            pltpu.async_copy(rt, out_slice(u), sem)

        def wait_store(u, rt, sem):
            pltpu.make_async_copy(rt, out_slice(u), sem).wait()

        load_gidx(0, gidx_a, sem_ia)
        load_gidx(1, gidx_b, sem_ib)
        wait_gidx(0, gidx_a, sem_ia)
        gather(gidx_a, rows_a, sem_ga)
        wait_gidx(1, gidx_b, sem_ib)
        gather(gidx_b, rows_b, sem_gb)

        def body(i, carry):
            u0 = 2 * i
            u1 = u0 + 1
            wait_gather(gidx_a, rows_a, sem_ga)

            @pl.when(u0 + 2 < N_UNITS)
            def _():
                load_gidx(u0 + 2, gidx_a, sem_ia)

            @pl.when(i > 0)
            def _():
                wait_store(u0 - 2, rt_a, sem_sa)

            transpose(rows_a, rt_a)
            store(u0, rt_a, sem_sa)

            @pl.when(u0 + 2 < N_UNITS)
            def _():
                wait_gidx(u0 + 2, gidx_a, sem_ia)
                gather(gidx_a, rows_a, sem_ga)

            wait_gather(gidx_b, rows_b, sem_gb)

            @pl.when(u1 + 2 < N_UNITS)
            def _():
                load_gidx(u1 + 2, gidx_b, sem_ib)

            @pl.when(i > 0)
            def _():
                wait_store(u1 - 2, rt_b, sem_sb)

            transpose(rows_b, rt_b)
            store(u1, rt_b, sem_sb)

            @pl.when(u1 + 2 < N_UNITS)
            def _():
                wait_gidx(u1 + 2, gidx_b, sem_ib)
                gather(gidx_b, rows_b, sem_gb)

            return carry

        lax.fori_loop(0, N_UNITS // 2, body, 0)
        wait_store(N_UNITS - 2, rt_a, sem_sa)
        wait_store(N_UNITS - 1, rt_b, sem_sb)

    return k(weight, idx_flat)


NV = 1000000
TBLK = 2048


def _tc_relayout_table(wt):
    grid = (NV + TBLK - 1) // TBLK

    def body(wt_ref, out_ref):
        x = wt_ref[...]
        y = jnp.transpose(x)
        out_ref[:, 0:D] = y
        out_ref[:, D:128] = jnp.zeros((TBLK, 128 - D), jnp.float32)

    return pl.pallas_call(
        body,
        grid=(grid,),
        in_specs=[pl.BlockSpec((D, TBLK), lambda i: (0, i))],
        out_specs=pl.BlockSpec((TBLK, 128), lambda i: (i, 0)),
        out_shape=jax.ShapeDtypeStruct((NV, 128), jnp.float32),
    )(wt)


def kernel(input, weight):
    idx_flat = input.T.reshape(-1).astype(jnp.int32) * 2
    w_pad = _tc_relayout_table(weight.T)
    w2m = jnp.reshape(w_pad, (2 * NV, D))
    out5 = _sc_embedding_gather(idx_flat, w2m)
    return jnp.transpose(out5, (2, 4, 0, 1, 3)).reshape(NB, NS, D)
